# baseline (speedup 1.0000x reference)
import functools

import jax
import jax.numpy as jnp
from jax import lax
from jax.experimental import pallas as pl
from jax.experimental.pallas import tpu as pltpu
from jax.experimental.pallas import tpu_sc as plsc

_D = 64
_V = 6
_GRP = 16


@functools.cache
def _build(NB: int, S: int):
    info = plsc.get_sparse_core_info()
    nw = info.num_cores * info.num_subcores
    nb_per_w = NB // nw
    n_grp = -(-S // _GRP)
    last = S - _GRP
    assert NB % nw == 0 and nb_per_w % 2 == 0 and S % 8 == 0
    mesh = plsc.VectorSubcoreMesh(core_axis_name="c", subcore_axis_name="s")

    @functools.partial(
        pl.kernel,
        mesh=mesh,
        out_type=jax.ShapeDtypeStruct((NB * S, _D), jnp.float32),
        scratch_types=[
            pltpu.VMEM((_V * _D,), jnp.float32),
            pltpu.VMEM((nb_per_w, S), jnp.int32),
            pltpu.VMEM((2, S, _D), jnp.float32),
            pltpu.SemaphoreType.DMA,
            pltpu.SemaphoreType.DMA,
        ],
        compiler_params=pltpu.CompilerParams(needs_layout_passes=False),
    )
    def k(tflat_hbm, ids_hbm, out_hbm, tflat_v, ids_v, rows_v, o0, o1):
        osem = (o0, o1)
        wid = lax.axis_index("s") * info.num_cores + lax.axis_index("c")
        pltpu.sync_copy(tflat_hbm, tflat_v)
        pltpu.sync_copy(ids_hbm.at[pl.ds(wid * nb_per_w, nb_per_w)], ids_v)
        iota = lax.iota(jnp.int32, _GRP)

        def body(g, carry):
            for b in range(2):
                c = 2 * g + b
                out_slice = out_hbm.at[pl.ds((wid * nb_per_w + c) * S, S)]

                @pl.when(g > 0)
                def _wait_prev():
                    pltpu.make_async_copy(rows_v.at[b], out_slice,
                                          osem[b]).wait()

                def grp(i, carry2):
                    col = jnp.minimum(i * _GRP, last)
                    v_ids = ids_v[c, pl.ds(col, _GRP)]
                    v_off = v_ids * _D
                    for r in range(_GRP):
                        bc = lax.gather(
                            v_off,
                            jnp.full((_GRP, 1), r, jnp.int32),
                            lax.GatherDimensionNumbers(
                                offset_dims=(), collapsed_slice_dims=(0,),
                                start_index_map=(0,)),
                            (1,),
                            mode=lax.GatherScatterMode.PROMISE_IN_BOUNDS)
                        row = col + r
                        for j in range(_D // _GRP):
                            vals = plsc.load_gather(
                                tflat_v, [bc + (iota + j * _GRP)])
                            rows_v[b, row, pl.ds(j * _GRP, _GRP)] = vals
                    return carry2

                lax.fori_loop(0, n_grp, grp, 0)
                pltpu.make_async_copy(rows_v.at[b], out_slice,
                                      osem[b]).start()
            return carry

        lax.fori_loop(0, nb_per_w // 2, body, 0)
        for b in range(2):
            c = nb_per_w - 2 + b
            out_slice = out_hbm.at[pl.ds((wid * nb_per_w + c) * S, S)]
            pltpu.make_async_copy(rows_v.at[b], out_slice, osem[b]).wait()

    return k


def kernel(tone_ids, embed_weight):
    b, s = tone_ids.shape
    out = _build(b, s)(embed_weight.reshape(-1), tone_ids.astype(jnp.int32))
    return out.reshape(b, s, _D)
